# baseline (speedup 1.0000x reference)
import functools

import jax
import jax.numpy as jnp
from jax import lax
from jax.experimental import pallas as pl
from jax.experimental.pallas import tpu as pltpu
from jax.experimental.pallas import tpu_sc as plsc

N = 1048576
NS = 16
CHUNK = N // NS
L = 16
NVEC = CHUNK // L
J = int((1 - 0.1) * N)
HSTRIDE = 2048
ABS_MASK = 0x7FFFFFFF
U = 10
NCOPY = 1

_LEVELS = ((21, None, 10), (10, 21, 11), (0, 10, 10))

_mesh = plsc.VectorSubcoreMesh(
    core_axis_name="c", subcore_axis_name="s", num_cores=2, num_subcores=NS
)


@functools.partial(
    pl.kernel,
    out_type=(
        jax.ShapeDtypeStruct((128, 8192), jnp.float32),
        jax.ShapeDtypeStruct((8192, 128), jnp.float32),
        jax.ShapeDtypeStruct((2, NS * HSTRIDE), jnp.int32),
    ),
    mesh=_mesh,
    compiler_params=pltpu.CompilerParams(needs_layout_passes=False),
    scratch_types=[
        pltpu.VMEM((512, 128), jnp.float32),
        pltpu.VMEM((NS * HSTRIDE,), jnp.int32),
        pltpu.VMEM((HSTRIDE,), jnp.int32),
        pltpu.SemaphoreType.DMA((4,)),
    ],
)
def _subnet_kernel(a_in, b_in, a_out, b_out, hsc,
                   data_v, hist_v, red_v, sems):
    c = lax.axis_index("c")
    s = lax.axis_index("s")
    lanes = lax.iota(jnp.int32, 16)
    zero16 = jnp.zeros((16,), jnp.int32)
    one16 = jnp.ones((16,), jnp.int32)

    def process(in_ref, out_ref, h_ref, rows, cols):
        Q = CHUNK // 4
        R4 = rows // 4
        dv2 = data_v if cols == 128 else data_v.reshape(rows, cols)
        ld = [
            pltpu.async_copy(
                in_ref.at[pl.ds(s * rows + q * R4, R4), :],
                dv2.at[pl.ds(q * R4, R4), :], sems.at[q])
            for q in range(4)
        ]

        jr = jnp.int32(J)
        prefix = jnp.int32(0)
        last16 = jnp.full((16,), 15, jnp.int32)
        for shift, pshift, bits in _LEVELS:
            width = 1 << bits

            @plsc.parallel_loop(0, width, step=16, unroll=U)
            def _(i):
                hist_v[pl.ds(i, 16)] = zero16

            pfx = prefix

            def scan_range(lo, hi):
                @plsc.parallel_loop(lo, hi, step=16, unroll=U)
                def _(off):
                    a = lax.bitcast_convert_type(
                        data_v[off >> 7, pl.ds(off & 127, 16)],
                        jnp.int32) & ABS_MASK
                    bn = ((a >> shift) & (width - 1) if shift
                          else a & (width - 1))
                    if pshift is None:
                        plsc.addupdate_scatter(hist_v, [bn], one16)
                    else:
                        m = (a >> pshift) == pfx
                        plsc.addupdate_scatter(hist_v, [bn], one16, mask=m)

            if pshift is None:
                for q in range(4):
                    ld[q].wait()
                    scan_range(q * Q, (q + 1) * Q)
            else:
                scan_range(0, CHUNK)

            def lred(ncopies):
                @plsc.parallel_loop(0, width, step=16, unroll=4)
                def _(r):
                    acc = hist_v[pl.ds(r, 16)]
                    for l in range(1, ncopies):
                        acc = acc + hist_v[pl.ds(l * width + r, 16)]
                    red_v[pl.ds(r, 16)] = acc

            pltpu.sync_copy(hist_v.at[pl.ds(0, width)],
                            h_ref.at[pl.ds(s * width, width)])
            plsc.subcore_barrier()
            pltpu.sync_copy(h_ref.at[pl.ds(0, NS * width)],
                            hist_v.at[pl.ds(0, NS * width)])
            lred(NS)

            z16 = jnp.zeros((16,), jnp.int32)

            @plsc.parallel_loop(0, width, step=16, unroll=2,
                                carry=(z16, z16))
            def crow_out(r, carry):
                tot, cnt = carry
                cf = tot + plsc.cumsum(red_v[pl.ds(r, 16)])
                red_v[pl.ds(r, 16)] = cf
                tot = cf.at[last16].get(mode="promise_in_bounds")
                cnt = cnt + jnp.where(cf <= jr, 1, 0)
                return (tot, cnt)

            _, cnt = crow_out
            fbin = jnp.sum(cnt)
            bm1 = jnp.maximum(fbin - 1, 0)
            vrow = red_v[pl.ds((bm1 >> 4) * 16, 16)]
            before_raw = jnp.max(jnp.where(lanes == (bm1 & 15), vrow, 0))
            before = jnp.where(fbin > 0, before_raw, 0)
            jr = jr - before
            prefix = (prefix << bits) | fbin

        t = prefix

        st = []
        for q in range(4):
            @plsc.parallel_loop(q * Q, (q + 1) * Q, step=16, unroll=U)
            def _(off):
                a = lax.bitcast_convert_type(
                    data_v[off >> 7, pl.ds(off & 127, 16)],
                    jnp.int32) & ABS_MASK
                data_v[off >> 7, pl.ds(off & 127, 16)] = jnp.where(
                    a >= t, 1.0, 0.0).astype(jnp.float32)
            st.append(pltpu.async_copy(
                dv2.at[pl.ds(q * R4, R4), :],
                out_ref.at[pl.ds(s * rows + q * R4, R4), :], sems.at[q]))
        for d in st:
            d.wait()

    @pl.when(c == 0)
    def _():
        process(a_in, a_out, hsc.at[0], 8, 8192)

    @pl.when(c == 1)
    def _():
        process(b_in, b_out, hsc.at[1], 512, 128)


def kernel(lora_A_mask, lora_B_mask):
    ma, mb, _ = _subnet_kernel(lora_A_mask, lora_B_mask)
    return ma, mb


# speedup vs baseline: 1.0238x; 1.0238x over previous
import functools

import jax
import jax.numpy as jnp
from jax import lax
from jax.experimental import pallas as pl
from jax.experimental.pallas import tpu as pltpu
from jax.experimental.pallas import tpu_sc as plsc

N = 1048576
NS = 16
CHUNK = N // NS
L = 16
NVEC = CHUNK // L
J = int((1 - 0.1) * N)
HSTRIDE = 2048
ABS_MASK = 0x7FFFFFFF
U = 8
NCOPY = 1

_LEVELS = ((21, None, 10), (10, 21, 11), (0, 10, 10))

_mesh = plsc.VectorSubcoreMesh(
    core_axis_name="c", subcore_axis_name="s", num_cores=2, num_subcores=NS
)


@functools.partial(
    pl.kernel,
    out_type=(
        jax.ShapeDtypeStruct((128, 8192), jnp.float32),
        jax.ShapeDtypeStruct((8192, 128), jnp.float32),
        jax.ShapeDtypeStruct((2, NS * HSTRIDE), jnp.int32),
    ),
    mesh=_mesh,
    compiler_params=pltpu.CompilerParams(needs_layout_passes=False),
    scratch_types=[
        pltpu.VMEM((512, 128), jnp.float32),
        pltpu.VMEM((NS * HSTRIDE,), jnp.int32),
        pltpu.VMEM((HSTRIDE,), jnp.int32),
        pltpu.SemaphoreType.DMA((4,)),
    ],
)
def _subnet_kernel(a_in, b_in, a_out, b_out, hsc,
                   data_v, hist_v, red_v, sems):
    c = lax.axis_index("c")
    s = lax.axis_index("s")
    lanes = lax.iota(jnp.int32, 16)
    zero16 = jnp.zeros((16,), jnp.int32)
    one16 = jnp.ones((16,), jnp.int32)

    def process(in_ref, out_ref, h_ref, rows, cols):
        Q = CHUNK // 4
        R4 = rows // 4
        dv2 = data_v if cols == 128 else data_v.reshape(rows, cols)
        ld = [
            pltpu.async_copy(
                in_ref.at[pl.ds(s * rows + q * R4, R4), :],
                dv2.at[pl.ds(q * R4, R4), :], sems.at[q])
            for q in range(4)
        ]

        jr = jnp.int32(J)
        prefix = jnp.int32(0)
        last16 = jnp.full((16,), 15, jnp.int32)
        for shift, pshift, bits in _LEVELS:
            width = 1 << bits

            @plsc.parallel_loop(0, width, step=16, unroll=U)
            def _(i):
                hist_v[pl.ds(i, 16)] = zero16

            pfx = prefix

            def scan_range(lo, hi):
                @plsc.parallel_loop(lo, hi, step=16, unroll=U)
                def _(off):
                    a = lax.bitcast_convert_type(
                        data_v[off >> 7, pl.ds(off & 127, 16)],
                        jnp.int32) & ABS_MASK
                    bn = ((a >> shift) & (width - 1) if shift
                          else a & (width - 1))
                    if pshift is None:
                        plsc.addupdate_scatter(hist_v, [bn], one16)
                    else:
                        m = (a >> pshift) == pfx
                        plsc.addupdate_scatter(hist_v, [bn], one16, mask=m)

            if pshift is None:
                for q in range(4):
                    ld[q].wait()
                    scan_range(q * Q, (q + 1) * Q)
            else:
                scan_range(0, CHUNK)

            def lred(ncopies):
                @plsc.parallel_loop(0, width, step=16, unroll=4)
                def _(r):
                    acc = hist_v[pl.ds(r, 16)]
                    for l in range(1, ncopies):
                        acc = acc + hist_v[pl.ds(l * width + r, 16)]
                    red_v[pl.ds(r, 16)] = acc

            pltpu.sync_copy(hist_v.at[pl.ds(0, width)],
                            h_ref.at[pl.ds(s * width, width)])
            plsc.subcore_barrier()
            pltpu.sync_copy(h_ref.at[pl.ds(0, NS * width)],
                            hist_v.at[pl.ds(0, NS * width)])
            lred(NS)

            z16 = jnp.zeros((16,), jnp.int32)

            @plsc.parallel_loop(0, width, step=16, unroll=2,
                                carry=(z16, z16))
            def crow_out(r, carry):
                tot, cnt = carry
                cf = tot + plsc.cumsum(red_v[pl.ds(r, 16)])
                red_v[pl.ds(r, 16)] = cf
                tot = cf.at[last16].get(mode="promise_in_bounds")
                cnt = cnt + jnp.where(cf <= jr, 1, 0)
                return (tot, cnt)

            _, cnt = crow_out
            fbin = jnp.sum(cnt)
            bm1 = jnp.maximum(fbin - 1, 0)
            vrow = red_v[pl.ds((bm1 >> 4) * 16, 16)]
            before_raw = jnp.max(jnp.where(lanes == (bm1 & 15), vrow, 0))
            before = jnp.where(fbin > 0, before_raw, 0)
            jr = jr - before
            prefix = (prefix << bits) | fbin

        t = prefix

        st = []
        for q in range(4):
            @plsc.parallel_loop(q * Q, (q + 1) * Q, step=16, unroll=U)
            def _(off):
                a = lax.bitcast_convert_type(
                    data_v[off >> 7, pl.ds(off & 127, 16)],
                    jnp.int32) & ABS_MASK
                data_v[off >> 7, pl.ds(off & 127, 16)] = jnp.where(
                    a >= t, 1.0, 0.0).astype(jnp.float32)
            st.append(pltpu.async_copy(
                dv2.at[pl.ds(q * R4, R4), :],
                out_ref.at[pl.ds(s * rows + q * R4, R4), :], sems.at[q]))
        for d in st:
            d.wait()

    @pl.when(c == 0)
    def _():
        process(a_in, a_out, hsc.at[0], 8, 8192)

    @pl.when(c == 1)
    def _():
        process(b_in, b_out, hsc.at[1], 512, 128)


def kernel(lora_A_mask, lora_B_mask):
    ma, mb, _ = _subnet_kernel(lora_A_mask, lora_B_mask)
    return ma, mb
